# Initial kernel scaffold; baseline (speedup 1.0000x reference)
import jax
import jax.numpy as jnp
from jax.experimental import pallas as pl


def kernel(z_mean, z_var, x, epsilon):
    raise NotImplementedError("write your pallas kernel here")


# speedup vs baseline: 1.1026x; 1.1026x over previous
import functools

import jax
import jax.numpy as jnp
from jax import lax
from jax.experimental import pallas as pl
from jax.experimental.pallas import tpu as pltpu
from jax.experimental.pallas import tpu_sc as plsc

SCALE_FACTOR = 0.9
ZERO_POINT = 1e-08

BATCH = 128
DIM = 64
NOUT = 8192
MULT = NOUT // BATCH
PROW = DIM * MULT
NW = 32
ROWS_PER_W = BATCH // NW
L = 16


def _reduce_lanes(vec, op):
    s = vec[0]
    for i in range(1, L):
        s = op(s, vec[i])
    return s


def _sc_hist_body(zm_hbm, zv_hbm, eps_hbm, cnt_hbm, zm_v, sc_v, eps_v, p_v, cnt_v):
    wid = lax.axis_index("s") * 2 + lax.axis_index("c")
    ones = jnp.ones((L,), jnp.float32)

    for rl in range(ROWS_PER_W):
        r = wid * ROWS_PER_W + rl
        pltpu.sync_copy(zm_hbm.at[pl.ds(r * DIM, DIM)], zm_v)
        pltpu.sync_copy(zv_hbm.at[pl.ds(r * DIM, DIM)], sc_v)
        pltpu.sync_copy(eps_hbm.at[pl.ds(r * PROW, PROW)], eps_v)

        for k in range(DIM // L):
            sc_v[pl.ds(k * L, L)] = jnp.exp(0.5 * sc_v[pl.ds(k * L, L)])

        big = jnp.full((L,), jnp.inf, jnp.float32)

        def samp_body(j, carry):
            vmin_c, vmax_c = carry
            koff = lax.rem(j, DIM // L) * L
            v = zm_v[pl.ds(koff, L)] + sc_v[pl.ds(koff, L)] * eps_v[pl.ds(j * L, L)]
            p_v[pl.ds(j * L, L)] = v
            return jnp.minimum(vmin_c, v), jnp.maximum(vmax_c, v)

        vmin_v, vmax_v = lax.fori_loop(0, PROW // L, samp_body, (big, -big))
        vmin = _reduce_lanes(vmin_v, jnp.minimum)
        vmax = _reduce_lanes(vmax_v, jnp.maximum)
        width = (vmax - vmin) * jnp.float32(1.0 / NOUT)
        width = jnp.where(width <= 0.0, jnp.float32(1.0), width)
        vmin_b = jnp.full((L,), vmin, jnp.float32)
        width_b = jnp.full((L,), width, jnp.float32)

        def zero_body(j, _):
            cnt_v[pl.ds(j * L, L)] = jnp.zeros((L,), jnp.float32)
            return 0

        lax.fori_loop(0, NOUT // L, zero_body, 0)

        def scat_body(j, _):
            v = p_v[pl.ds(j * L, L)]
            idx = ((v - vmin_b) / width_b).astype(jnp.int32)
            idx = jnp.clip(idx, 0, NOUT - 1)
            plsc.addupdate_scatter(cnt_v, [idx], ones)
            return 0

        lax.fori_loop(0, PROW // L, scat_body, 0)

        pltpu.sync_copy(cnt_v, cnt_hbm.at[pl.ds(r * NOUT, NOUT)])


def _sc_hist(zm_flat, zv_flat, eps_flat):
    mesh = plsc.VectorSubcoreMesh(core_axis_name="c", subcore_axis_name="s")
    f = functools.partial(
        pl.kernel,
        _sc_hist_body,
        mesh=mesh,
        out_type=jax.ShapeDtypeStruct((BATCH * NOUT,), jnp.float32),
        scratch_types=[
            pltpu.VMEM((DIM,), jnp.float32),
            pltpu.VMEM((DIM,), jnp.float32),
            pltpu.VMEM((PROW,), jnp.float32),
            pltpu.VMEM((PROW,), jnp.float32),
            pltpu.VMEM((NOUT,), jnp.float32),
        ],
        compiler_params=pltpu.CompilerParams(needs_layout_passes=False),
    )()
    return f(zm_flat, zv_flat, eps_flat)


def _tc_softmax_mul_body(cnt_ref, x_ref, o_ref):
    c = cnt_ref[...]
    m = jnp.max(c, axis=1, keepdims=True)
    e = jnp.exp(c - m)
    s = jnp.sum(e, axis=1, keepdims=True)
    p = e / s
    p = jnp.where(p < ZERO_POINT, jnp.zeros_like(p), p)
    o_ref[...] = x_ref[...] * p / SCALE_FACTOR


def _tc_softmax_mul(counts, x):
    rb = 16
    return pl.pallas_call(
        _tc_softmax_mul_body,
        grid=(BATCH // rb,),
        in_specs=[
            pl.BlockSpec((rb, NOUT), lambda i: (i, 0)),
            pl.BlockSpec((rb, NOUT), lambda i: (i, 0)),
        ],
        out_specs=pl.BlockSpec((rb, NOUT), lambda i: (i, 0)),
        out_shape=jax.ShapeDtypeStruct((BATCH, NOUT), jnp.float32),
    )(counts, x)


def kernel(z_mean, z_var, x, epsilon):
    counts = _sc_hist(
        z_mean.reshape(-1), z_var.reshape(-1), epsilon.reshape(-1)
    ).reshape(BATCH, NOUT)
    return _tc_softmax_mul(counts, x)


# speedup vs baseline: 1.3270x; 1.2034x over previous
import functools

import jax
import jax.numpy as jnp
from jax import lax
from jax.experimental import pallas as pl
from jax.experimental.pallas import tpu as pltpu
from jax.experimental.pallas import tpu_sc as plsc

SCALE_FACTOR = 0.9
ZERO_POINT = 1e-08

BATCH = 128
DIM = 64
NOUT = 8192
MULT = NOUT // BATCH
PROW = DIM * MULT
NW = 32
ROWS_PER_W = BATCH // NW
L = 16


def _reduce_lanes(vec, op):
    s = vec[0]
    for i in range(1, L):
        s = op(s, vec[i])
    return s


def _sc_hist_body(zm_hbm, zv_hbm, eps_hbm, cnt_hbm, zm_v, sc_v, eps_v, p_v, cnt_v):
    wid = lax.axis_index("s") * 2 + lax.axis_index("c")
    ones = jnp.ones((L,), jnp.float32)

    for rl in range(ROWS_PER_W):
        r = wid * ROWS_PER_W + rl
        pltpu.sync_copy(zm_hbm.at[pl.ds(r * DIM, DIM)], zm_v)
        pltpu.sync_copy(zv_hbm.at[pl.ds(r * DIM, DIM)], sc_v)
        pltpu.sync_copy(eps_hbm.at[pl.ds(r * PROW, PROW)], eps_v)

        scs = [jnp.exp(0.5 * sc_v[pl.ds(k * L, L)]) for k in range(DIM // L)]
        zms = [zm_v[pl.ds(k * L, L)] for k in range(DIM // L)]

        big = jnp.full((L,), jnp.inf, jnp.float32)
        KU = DIM // L

        def samp_body(e, carry):
            mins, maxs = carry
            nmins, nmaxs = [], []
            for k in range(KU):
                off = e * DIM + k * L
                v = zms[k] + scs[k] * eps_v[pl.ds(off, L)]
                p_v[pl.ds(off, L)] = v
                nmins.append(jnp.minimum(mins[k], v))
                nmaxs.append(jnp.maximum(maxs[k], v))
            return tuple(nmins), tuple(nmaxs)

        mins, maxs = lax.fori_loop(
            0, MULT, samp_body, ((big,) * KU, (-big,) * KU)
        )
        vmin_v = jnp.minimum(jnp.minimum(mins[0], mins[1]), jnp.minimum(mins[2], mins[3]))
        vmax_v = jnp.maximum(jnp.maximum(maxs[0], maxs[1]), jnp.maximum(maxs[2], maxs[3]))
        vmin = _reduce_lanes(vmin_v, jnp.minimum)
        vmax = _reduce_lanes(vmax_v, jnp.maximum)
        width = (vmax - vmin) * jnp.float32(1.0 / NOUT)
        width = jnp.where(width <= 0.0, jnp.float32(1.0), width)
        vmin_b = jnp.full((L,), vmin, jnp.float32)
        inv_w_b = jnp.float32(1.0) / jnp.full((L,), width, jnp.float32)

        ZU = 16
        zero = jnp.zeros((L,), jnp.float32)

        def zero_body(j, _):
            for k in range(ZU):
                cnt_v[pl.ds((j * ZU + k) * L, L)] = zero
            return 0

        lax.fori_loop(0, NOUT // L // ZU, zero_body, 0)

        SU = 8

        def scat_body(j, _):
            for k in range(SU):
                v = p_v[pl.ds((j * SU + k) * L, L)]
                idx = ((v - vmin_b) * inv_w_b).astype(jnp.int32)
                idx = jnp.clip(idx, 0, NOUT - 1)
                plsc.addupdate_scatter(cnt_v, [idx], ones)
            return 0

        lax.fori_loop(0, PROW // L // SU, scat_body, 0)

        pltpu.sync_copy(cnt_v, cnt_hbm.at[pl.ds(r * NOUT, NOUT)])


def _sc_hist(zm_flat, zv_flat, eps_flat):
    mesh = plsc.VectorSubcoreMesh(core_axis_name="c", subcore_axis_name="s")
    f = functools.partial(
        pl.kernel,
        _sc_hist_body,
        mesh=mesh,
        out_type=jax.ShapeDtypeStruct((BATCH * NOUT,), jnp.float32),
        scratch_types=[
            pltpu.VMEM((DIM,), jnp.float32),
            pltpu.VMEM((DIM,), jnp.float32),
            pltpu.VMEM((PROW,), jnp.float32),
            pltpu.VMEM((PROW,), jnp.float32),
            pltpu.VMEM((NOUT,), jnp.float32),
        ],
        compiler_params=pltpu.CompilerParams(needs_layout_passes=False),
    )()
    return f(zm_flat, zv_flat, eps_flat)


def _tc_softmax_mul_body(cnt_ref, x_ref, o_ref):
    c = cnt_ref[...]
    m = jnp.max(c, axis=1, keepdims=True)
    e = jnp.exp(c - m)
    s = jnp.sum(e, axis=1, keepdims=True)
    p = e / s
    p = jnp.where(p < ZERO_POINT, jnp.zeros_like(p), p)
    o_ref[...] = x_ref[...] * p / SCALE_FACTOR


def _tc_softmax_mul(counts, x):
    rb = 16
    return pl.pallas_call(
        _tc_softmax_mul_body,
        grid=(BATCH // rb,),
        in_specs=[
            pl.BlockSpec((rb, NOUT), lambda i: (i, 0)),
            pl.BlockSpec((rb, NOUT), lambda i: (i, 0)),
        ],
        out_specs=pl.BlockSpec((rb, NOUT), lambda i: (i, 0)),
        out_shape=jax.ShapeDtypeStruct((BATCH, NOUT), jnp.float32),
    )(counts, x)


def kernel(z_mean, z_var, x, epsilon):
    counts = _sc_hist(
        z_mean.reshape(-1), z_var.reshape(-1), epsilon.reshape(-1)
    ).reshape(BATCH, NOUT)
    return _tc_softmax_mul(counts, x)


# speedup vs baseline: 1.7313x; 1.3048x over previous
import jax
import jax.numpy as jnp
from jax import lax
from jax.experimental import pallas as pl
from jax.experimental.pallas import tpu as pltpu
from jax.experimental.pallas import tpu_sc as plsc

SCALE_FACTOR = 0.9
ZERO_POINT = 1e-08

BATCH = 128
DIM = 64
NOUT = 8192
MULT = NOUT // BATCH
PROW = DIM * MULT
NW = 32
ROWS_PER_W = BATCH // NW
L = 16


def _reduce_lanes(vec, op):
    s = vec[0]
    for i in range(1, L):
        s = op(s, vec[i])
    return s


def _sc_hist_body(zm_hbm, zv_hbm, eps_hbm, cnt_hbm, zm_v, zv_v, eps_v, p_v, cnt_v):
    wid = lax.axis_index("s") * 2 + lax.axis_index("c")
    ones = jnp.ones((L,), jnp.float32)

    for rl in range(ROWS_PER_W):
        r = wid * ROWS_PER_W + rl
        pltpu.sync_copy(zm_hbm.at[pl.ds(r, 1)], zm_v)
        pltpu.sync_copy(zv_hbm.at[pl.ds(r, 1)], zv_v)
        pltpu.sync_copy(eps_hbm.at[pl.ds(r * MULT, MULT)], eps_v)

        scs = [jnp.exp(0.5 * zv_v[0, pl.ds(k * L, L)]) for k in range(DIM // L)]
        zms = [zm_v[0, pl.ds(k * L, L)] for k in range(DIM // L)]

        big = jnp.full((L,), jnp.inf, jnp.float32)
        KU = DIM // L

        def samp_body(e, carry):
            mins, maxs = carry
            nmins, nmaxs = [], []
            for k in range(KU):
                v = zms[k] + scs[k] * eps_v[e, pl.ds(k * L, L)]
                p_v[pl.ds(e * DIM + k * L, L)] = v
                nmins.append(jnp.minimum(mins[k], v))
                nmaxs.append(jnp.maximum(maxs[k], v))
            return tuple(nmins), tuple(nmaxs)

        mins, maxs = lax.fori_loop(
            0, MULT, samp_body, ((big,) * KU, (-big,) * KU)
        )
        vmin_v = jnp.minimum(jnp.minimum(mins[0], mins[1]), jnp.minimum(mins[2], mins[3]))
        vmax_v = jnp.maximum(jnp.maximum(maxs[0], maxs[1]), jnp.maximum(maxs[2], maxs[3]))
        vmin = _reduce_lanes(vmin_v, jnp.minimum)
        vmax = _reduce_lanes(vmax_v, jnp.maximum)
        width = (vmax - vmin) * jnp.float32(1.0 / NOUT)
        width = jnp.where(width <= 0.0, jnp.float32(1.0), width)
        vmin_b = jnp.full((L,), vmin, jnp.float32)
        inv_w_b = jnp.float32(1.0) / jnp.full((L,), width, jnp.float32)

        ZU = 16
        zero = jnp.zeros((L,), jnp.float32)

        def zero_body(j, _):
            for k in range(ZU):
                cnt_v[pl.ds((j * ZU + k) * L, L)] = zero
            return 0

        lax.fori_loop(0, NOUT // L // ZU, zero_body, 0)

        SU = 8

        def scat_body(j, _):
            for k in range(SU):
                v = p_v[pl.ds((j * SU + k) * L, L)]
                idx = ((v - vmin_b) * inv_w_b).astype(jnp.int32)
                idx = jnp.clip(idx, 0, NOUT - 1)
                plsc.addupdate_scatter(cnt_v, [idx], ones)
            return 0

        lax.fori_loop(0, PROW // L // SU, scat_body, 0)

        pltpu.sync_copy(cnt_v, cnt_hbm.at[r])


def _sc_hist(z_mean, z_var, epsilon):
    mesh = plsc.VectorSubcoreMesh(core_axis_name="c", subcore_axis_name="s")
    f = pl.kernel(
        _sc_hist_body,
        mesh=mesh,
        out_type=jax.ShapeDtypeStruct((BATCH, NOUT), jnp.float32),
        scratch_types=[
            pltpu.VMEM((1, DIM), jnp.float32),
            pltpu.VMEM((1, DIM), jnp.float32),
            pltpu.VMEM((MULT, DIM), jnp.float32),
            pltpu.VMEM((PROW,), jnp.float32),
            pltpu.VMEM((NOUT,), jnp.float32),
        ],
        compiler_params=pltpu.CompilerParams(needs_layout_passes=False),
    )
    return f(z_mean, z_var, epsilon)


def _tc_softmax_mul_body(cnt_ref, x_ref, o_ref):
    c = cnt_ref[...]
    m = jnp.max(c, axis=1, keepdims=True)
    e = jnp.exp(c - m)
    s = jnp.sum(e, axis=1, keepdims=True)
    p = e / s
    p = jnp.where(p < ZERO_POINT, jnp.zeros_like(p), p)
    o_ref[...] = x_ref[...] * p / SCALE_FACTOR


def _tc_softmax_mul(counts, x):
    rb = 16
    return pl.pallas_call(
        _tc_softmax_mul_body,
        grid=(BATCH // rb,),
        in_specs=[
            pl.BlockSpec((rb, NOUT), lambda i: (i, 0)),
            pl.BlockSpec((rb, NOUT), lambda i: (i, 0)),
        ],
        out_specs=pl.BlockSpec((rb, NOUT), lambda i: (i, 0)),
        out_shape=jax.ShapeDtypeStruct((BATCH, NOUT), jnp.float32),
    )(counts, x)


def kernel(z_mean, z_var, x, epsilon):
    counts = _sc_hist(z_mean, z_var, epsilon)
    return _tc_softmax_mul(counts, x)
